# baseline (speedup 1.0000x reference)
import functools

import jax
import jax.numpy as jnp
from jax import lax
from jax.experimental import pallas as pl
from jax.experimental.pallas import tpu as pltpu
from jax.experimental.pallas import tpu_sc as plsc

VOCAB = 100000
DIM = 128
BATCH = 16384
NEG = 20
CLAMP = 4.0

NCORE = 2
NSUB = 16
NWORK = NCORE * NSUB
EPW = BATCH // NWORK
CB = 16
NCHUNK = EPW // CB
ROWS_PER_E = NEG + 1
CNEG = CB * NEG

POLY = (
    3.9567963310604193e-10,
    -2.913936634074266e-08,
    9.670274319019722e-07,
    -2.0176458545796827e-05,
    0.0003274348917812428,
    -0.005175279938436231,
    0.12497826129850023,
    0.6931495678406687,
)

_mesh = plsc.VectorSubcoreMesh(core_axis_name="c", subcore_axis_name="s")

_GDN = lax.GatherDimensionNumbers(
    offset_dims=(), collapsed_slice_dims=(0,), start_index_map=(0,))


def _shuffle(x, idx):
    return lax.gather(x, idx[:, None], dimension_numbers=_GDN,
                      slice_sizes=(1,),
                      mode=lax.GatherScatterMode.PROMISE_IN_BOUNDS)


@functools.partial(
    pl.kernel,
    mesh=_mesh,
    out_type=jax.ShapeDtypeStruct((BATCH,), jnp.float32),
    scratch_types=[
        pltpu.VMEM((EPW,), jnp.int32),
        pltpu.VMEM((EPW,), jnp.int32),
        pltpu.VMEM((EPW * NEG,), jnp.int32),
        pltpu.VMEM((2, CB, DIM), jnp.float32),
        pltpu.VMEM((2, CB, DIM), jnp.float32),
        pltpu.VMEM((2, CNEG, DIM), jnp.float32),
        pltpu.VMEM((CB, 32), jnp.float32),
        pltpu.VMEM((2, CB), jnp.float32),
        pltpu.SemaphoreType.DMA,
        pltpu.SemaphoreType.DMA,
        pltpu.SemaphoreType.DMA,
        pltpu.SemaphoreType.DMA,
    ],
)
def _sc_loss(in_idx_hbm, pos_idx_hbm, neg_idx_hbm, emb_in_hbm, emb_out_hbm,
             loss_hbm, inidx_w, posidx_w, negidx_w, inrows2, posrows2,
             negrows2, score, lossbuf2, semA, semB, semW0, semW1):
    cid = lax.axis_index("c")
    sid = lax.axis_index("s")
    wid = sid * NCORE + cid
    ebase = wid * EPW
    pltpu.sync_copy(in_idx_hbm.at[pl.ds(ebase, EPW)], inidx_w)
    pltpu.sync_copy(pos_idx_hbm.at[pl.ds(ebase, EPW)], posidx_w)
    pltpu.sync_copy(neg_idx_hbm.at[pl.ds(ebase * NEG, EPW * NEG)], negidx_w)
    lane = lax.broadcasted_iota(jnp.int32, (16,), 0)
    perms = [lane ^ 8, lane ^ 4, lane ^ 2, lane ^ 1]

    bufs = (
        (inrows2.at[0], posrows2.at[0], negrows2.at[0], semA),
        (inrows2.at[1], posrows2.at[1], negrows2.at[1], semB),
    )
    wbufs = ((lossbuf2.at[0], semW0), (lossbuf2.at[1], semW1))

    def wb_copy(c, wbuf):
        lb, semW = wbuf
        return pltpu.make_async_copy(
            lb, loss_hbm.at[pl.ds(ebase + c * CB, CB)], semW)

    def copies(c, buf):
        inb, posb, negb, sem = buf
        cps = [
            pltpu.make_async_copy(
                emb_in_hbm.at[inidx_w.at[pl.ds(c * CB, CB)]], inb, sem),
            pltpu.make_async_copy(
                emb_out_hbm.at[posidx_w.at[pl.ds(c * CB, CB)]], posb, sem),
        ]
        for g, ln in ((0, 128), (128, 128), (256, 64)):
            cps.append(pltpu.make_async_copy(
                emb_out_hbm.at[negidx_w.at[pl.ds(c * CNEG + g, ln)]],
                negb.at[pl.ds(g, ln)], sem))
        return cps

    def issue(c, buf):
        for cp in copies(c, buf):
            cp.start()

    def drain(c, buf):
        for cp in copies(c, buf):
            cp.wait()

    def compute(c, buf, wbuf):
        inb, posb, negb, _ = buf
        lb, _ = wbuf

        def one_elem(e):
            vin = [inb[e, pl.ds(k * 16, 16)] for k in range(8)]
            srow0 = jnp.zeros((16,), jnp.float32)
            srow1 = jnp.zeros((16,), jnp.float32)
            for j in range(ROWS_PER_E):
                if j == 0:
                    row = posb.at[e]
                else:
                    row = negb.at[e * NEG + (j - 1)]
                acc = vin[0] * row[pl.ds(0, 16)]
                for k in range(1, 8):
                    acc = acc + vin[k] * row[pl.ds(k * 16, 16)]
                for p in perms:
                    acc = acc + _shuffle(acc, p)
                t = jnp.clip(acc, -CLAMP, CLAMP)
                if j == 0:
                    t = -t
                if j < 16:
                    srow0 = jnp.where(lane == j, t, srow0)
                else:
                    srow1 = jnp.where(lane == (j - 16), t, srow1)
            score[e, pl.ds(0, 16)] = srow0
            score[e, pl.ds(16, 16)] = srow1

        def elem2(i2, carry2):
            one_elem(2 * i2)
            one_elem(2 * i2 + 1)
            return carry2

        lax.fori_loop(0, CB // 2, elem2, 0)

        def softplus_vec(t_):
            u = t_ * t_
            pacc = jnp.full((16,), POLY[0], jnp.float32)
            for co in POLY[1:]:
                pacc = pacc * u + co
            return 0.5 * t_ + pacc

        def finish(e, lossrow):
            v = softplus_vec(score[e, pl.ds(0, 16)]) + jnp.where(
                lane < ROWS_PER_E - 16,
                softplus_vec(score[e, pl.ds(16, 16)]), 0.0)
            for p in perms:
                v = v + _shuffle(v, p)
            return jnp.where(lane == e, v, lossrow)

        lossrow = lax.fori_loop(0, CB, finish, jnp.zeros((16,), jnp.float32))
        pl.when(c >= 2)(lambda: wb_copy(c - 2, wbuf).wait())
        lb[...] = lossrow
        wb_copy(c, wbuf).start()

    issue(0, bufs[0])

    def body2(i, carry):
        c0 = 2 * i
        issue(c0 + 1, bufs[1])
        drain(c0, bufs[0])
        compute(c0, bufs[0], wbufs[0])
        pl.when(i < NCHUNK // 2 - 1)(lambda: issue(c0 + 2, bufs[0]))
        drain(c0 + 1, bufs[1])
        compute(c0 + 1, bufs[1], wbufs[1])
        return carry

    lax.fori_loop(0, NCHUNK // 2, body2, 0)
    wb_copy(NCHUNK - 2, wbufs[0]).wait()
    wb_copy(NCHUNK - 1, wbufs[1]).wait()


def kernel(inputs, positiveOutputs, negativeOutputs, emb_in, emb_out):
    return _sc_loss(inputs.astype(jnp.int32),
                    positiveOutputs.astype(jnp.int32),
                    negativeOutputs.astype(jnp.int32).reshape(-1),
                    emb_in, emb_out)
